# Initial kernel scaffold; baseline (speedup 1.0000x reference)
import jax
import jax.numpy as jnp
from jax.experimental import pallas as pl


def kernel(x, edge_index, W_gat, att_src, att_dst, b_gat, W_gcn, b_gcn, ln_gamma, ln_beta, fc2_W, fc2_b):
    raise NotImplementedError("write your pallas kernel here")


# speedup vs baseline: 1.0524x; 1.0524x over previous
import jax
import jax.numpy as jnp
from jax.experimental import pallas as pl
from jax.experimental.pallas import tpu as pltpu

_N = 10000
_D = 128
_C = 8
_OUT = 2


def _tail_body(comm_ref, xenc_ref, w3_ref, g_ref, b_ref, fcb_ref,
               allcomm_ref, probs_ref):
    comm = comm_ref[...]
    comm = jnp.where(comm >= 0, comm, 0.01 * comm)
    mu = jnp.mean(comm, axis=1, keepdims=True)
    var = jnp.mean((comm - mu) ** 2, axis=1, keepdims=True)
    y = (comm - mu) * jax.lax.rsqrt(var + 1e-5) * g_ref[...] + b_ref[...]
    y = y - jnp.max(y, axis=1, keepdims=True)
    ey = jnp.exp(y)
    allcomm_ref[...] = ey / jnp.sum(ey, axis=1, keepdims=True)

    xenc = xenc_ref[...]
    logits = jnp.sum(w3_ref[...] * xenc[None, :, :], axis=(1, 2)) + fcb_ref[0, :]
    logits = logits - jnp.max(logits)
    el = jnp.exp(logits)
    probs_ref[...] = (el / jnp.sum(el))[None, :]


def _tail(comm_pre, x_enc, fc2_W, fc2_b, ln_gamma, ln_beta):
    w3 = fc2_W.reshape(_OUT, _N, _D)
    allcomm, probs = pl.pallas_call(
        _tail_body,
        out_shape=(
            jax.ShapeDtypeStruct((_N, _C), jnp.float32),
            jax.ShapeDtypeStruct((1, _OUT), jnp.float32),
        ),
    )(comm_pre, x_enc, w3, ln_gamma.reshape(1, _C), ln_beta.reshape(1, _C),
      fc2_b.reshape(1, _OUT))
    return probs, allcomm.reshape(1, _N, _C)


def kernel(x, edge_index, W_gat, att_src, att_dst, b_gat, W_gcn, b_gcn,
           ln_gamma, ln_beta, fc2_W, fc2_b):
    loops = jnp.arange(_N, dtype=edge_index.dtype)
    src = jnp.concatenate([edge_index[0], loops])
    dst = jnp.concatenate([edge_index[1], loops])
    xw = x @ W_gat
    a_s = xw @ att_src
    a_d = xw @ att_dst
    alpha = jax.nn.leaky_relu(a_s[src] + a_d[dst], 0.2)
    ex = jnp.exp(alpha)
    denom = jax.ops.segment_sum(ex, dst, num_segments=_N)
    attn = ex / (denom[dst] + 1e-16)
    x_enc = jax.ops.segment_sum(attn[:, None] * xw[src], dst, num_segments=_N) + b_gat
    x_enc = jax.nn.elu(x_enc)
    deg = denom / (denom + 1e-16) + 1.0
    dis = jax.lax.rsqrt(deg)
    norm = dis[src] * attn * dis[dst]
    h = x_enc @ W_gcn
    comm_pre = jax.ops.segment_sum(norm[:, None] * h[src], dst, num_segments=_N)
    comm_pre = comm_pre + (1.0 / deg)[:, None] * h + b_gcn
    return _tail(comm_pre, x_enc, fc2_W, fc2_b, ln_gamma, ln_beta)


# speedup vs baseline: 2.7990x; 2.6596x over previous
import functools

import jax
import jax.numpy as jnp
from jax import lax
from jax.experimental import pallas as pl
from jax.experimental.pallas import tpu as pltpu
from jax.experimental.pallas import tpu_sc as plsc

_N = 10000
_D = 128
_C = 8
_OUT = 2
_E = 320000

_NC = 2
_NS = 16
_NW = _NC * _NS
_L = 16
_EPT = _E // _NW
_GRP = _EPT // _L

_sc_mesh = plsc.VectorSubcoreMesh(core_axis_name="c", subcore_axis_name="s")
_SC_PARAMS = pltpu.CompilerParams(
    use_tc_tiling_on_sc=False, needs_layout_passes=False)


@functools.partial(
    pl.kernel,
    out_type=jax.ShapeDtypeStruct((_NW, _N), jnp.float32),
    mesh=_sc_mesh,
    compiler_params=_SC_PARAMS,
    scratch_types=[
        pltpu.VMEM((_EPT,), jnp.int32),
        pltpu.VMEM((_EPT,), jnp.int32),
        pltpu.VMEM((_N,), jnp.float32),
        pltpu.VMEM((_N,), jnp.float32),
        pltpu.VMEM((_N,), jnp.float32),
    ],
)
def _denom_sc(src_hbm, dst_hbm, as_hbm, ad_hbm, out_hbm,
              src_v, dst_v, as_v, ad_v, acc_v):
    wid = lax.axis_index("s") * _NC + lax.axis_index("c")
    base = wid * _EPT
    pltpu.sync_copy(src_hbm.at[pl.ds(base, _EPT)], src_v)
    pltpu.sync_copy(dst_hbm.at[pl.ds(base, _EPT)], dst_v)
    pltpu.sync_copy(as_hbm, as_v)
    pltpu.sync_copy(ad_hbm, ad_v)

    zeros = jnp.zeros((_L,), jnp.float32)

    def zbody(i, c):
        acc_v[pl.ds(i * _L, _L)] = zeros
        return c

    lax.fori_loop(0, _N // _L, zbody, 0)

    def body(g, c):
        sv = src_v[pl.ds(g * _L, _L)]
        dv = dst_v[pl.ds(g * _L, _L)]
        t = plsc.load_gather(as_v, [sv]) + plsc.load_gather(ad_v, [dv])
        ex = jnp.exp(jnp.where(t >= 0, t, 0.2 * t))
        plsc.addupdate_scatter(acc_v, [dv], ex)
        return c

    lax.fori_loop(0, _GRP, body, 0)
    pltpu.sync_copy(acc_v, out_hbm.at[wid])


_QW = _D // 4
_QV = _QW // _L
_RB = 80
_EPT2 = _E // _NS
_GRP2 = _EPT2 // _L
_NB2 = _EPT2 // _RB
_RPS = _N // _NS
_NZR = _RPS // 5


def _build_xq(first):
    if first:
        out_type = (
            jax.ShapeDtypeStruct((_E,), jnp.float32),
            jax.ShapeDtypeStruct((_NC, _N, _QW), jnp.float32),
        )
        tab_scratch = [
            pltpu.VMEM((_N,), jnp.float32),
            pltpu.VMEM((_N,), jnp.float32),
            pltpu.VMEM((_N,), jnp.float32),
        ]
    else:
        out_type = jax.ShapeDtypeStruct((_NC, _N, _QW), jnp.float32)
        tab_scratch = []

    @functools.partial(
        pl.kernel,
        out_type=out_type,
        mesh=_sc_mesh,
        compiler_params=_SC_PARAMS,
        scratch_types=[
            pltpu.VMEM((_EPT2,), jnp.int32),
            pltpu.VMEM((_EPT2,), jnp.int32),
            *tab_scratch,
            pltpu.VMEM((_EPT2,), jnp.float32),
            pltpu.VMEM((_RB, _QW), jnp.float32),
            pltpu.VMEM((_NZR, _QW), jnp.float32),
            pltpu.VMEM_SHARED((_N, _QW), jnp.float32),
            pltpu.SemaphoreType.DMA,
        ],
    )
    def xq(*refs):
        if first:
            (src_hbm, dst_hbm, as_hbm, ad_hbm, den_hbm, tab_hbm,
             attn_hbm, cols_hbm,
             src_v, dst_v, as_v, ad_v, den_v, attn_v, rows_v, zbuf_v,
             acc_sh, sem) = refs
        else:
            (src_hbm, dst_hbm, attn_hbm, tab_hbm, cols_hbm,
             src_v, dst_v, attn_v, rows_v, zbuf_v, acc_sh, sem) = refs
        cid = lax.axis_index("c")
        sid = lax.axis_index("s")
        base = sid * _EPT2
        pltpu.sync_copy(src_hbm.at[pl.ds(base, _EPT2)], src_v)
        pltpu.sync_copy(dst_hbm.at[pl.ds(base, _EPT2)], dst_v)
        if first:
            pltpu.sync_copy(as_hbm, as_v)
            pltpu.sync_copy(ad_hbm, ad_v)
            pltpu.sync_copy(den_hbm, den_v)
        else:
            pltpu.sync_copy(attn_hbm.at[pl.ds(base, _EPT2)], attn_v)

        zrow = jnp.zeros((_L,), jnp.float32)
        for r in range(_NZR):
            for c in range(_QV):
                zbuf_v[r, pl.ds(c * _L, _L)] = zrow
        for k in range(5):
            pltpu.sync_copy(
                zbuf_v, acc_sh.at[pl.ds(sid * _RPS + k * _NZR, _NZR)])

        coff = cid * _N

        def prep_body(g, c):
            sv = src_v[pl.ds(g * _L, _L)]
            if first:
                dv = dst_v[pl.ds(g * _L, _L)]
                t = (plsc.load_gather(as_v, [sv])
                     + plsc.load_gather(ad_v, [dv]))
                ex = jnp.exp(jnp.where(t >= 0, t, 0.2 * t))
                attn_v[pl.ds(g * _L, _L)] = (
                    ex / (plsc.load_gather(den_v, [dv]) + 1e-16))
            src_v[pl.ds(g * _L, _L)] = sv + coff
            return c

        lax.fori_loop(0, _GRP2, prep_body, 0)

        if first:
            @pl.when(cid == 0)
            def _():
                pltpu.sync_copy(attn_v, attn_hbm.at[pl.ds(base, _EPT2)])

        plsc.subcore_barrier()

        def batch_body(b, c):
            eb = b * _RB
            pltpu.async_copy(
                tab_hbm.at[src_v.at[pl.ds(eb, _RB)]], rows_v, sem).wait()
            for g in range(_RB // _L):
                av = attn_v[pl.ds(eb + g * _L, _L)]
                for r16 in range(_L):
                    r = g * _L + r16
                    a = av[r16]
                    for cc in range(_QV):
                        rows_v[r, pl.ds(cc * _L, _L)] = (
                            rows_v[r, pl.ds(cc * _L, _L)] * a)
                dv = dst_v[pl.ds(eb + g * _L, _L)]
                pltpu.sync_copy(rows_v.at[pl.ds(g * _L, _L)], acc_sh.at[dv],
                                add=True)
            return c

        lax.fori_loop(0, _NB2, batch_body, 0)
        plsc.subcore_barrier()

        for k in range(5):
            off = sid * _RPS + k * _NZR
            pltpu.sync_copy(acc_sh.at[pl.ds(off, _NZR)], zbuf_v)
            pltpu.sync_copy(zbuf_v, cols_hbm.at[cid, pl.ds(off, _NZR)])

    return xq


_xq_first = _build_xq(True)
_xq_second = _build_xq(False)


def _tail_body(comm_ref, xenc_ref, w3_ref, g_ref, b_ref, fcb_ref,
               allcomm_ref, probs_ref):
    comm = comm_ref[...]
    comm = jnp.where(comm >= 0, comm, 0.01 * comm)
    mu = jnp.mean(comm, axis=1, keepdims=True)
    var = jnp.mean((comm - mu) ** 2, axis=1, keepdims=True)
    y = (comm - mu) * jax.lax.rsqrt(var + 1e-5) * g_ref[...] + b_ref[...]
    y = y - jnp.max(y, axis=1, keepdims=True)
    ey = jnp.exp(y)
    allcomm_ref[...] = ey / jnp.sum(ey, axis=1, keepdims=True)

    xenc = xenc_ref[...]
    logits = jnp.sum(w3_ref[...] * xenc[None, :, :], axis=(1, 2)) + fcb_ref[0, :]
    logits = logits - jnp.max(logits)
    el = jnp.exp(logits)
    probs_ref[...] = (el / jnp.sum(el))[None, :]


def _tail(comm_pre, x_enc, fc2_W, fc2_b, ln_gamma, ln_beta):
    w3 = fc2_W.reshape(_OUT, _N, _D)
    allcomm, probs = pl.pallas_call(
        _tail_body,
        out_shape=(
            jax.ShapeDtypeStruct((_N, _C), jnp.float32),
            jax.ShapeDtypeStruct((1, _OUT), jnp.float32),
        ),
    )(comm_pre, x_enc, w3, ln_gamma.reshape(1, _C), ln_beta.reshape(1, _C),
      fc2_b.reshape(1, _OUT))
    return probs, allcomm.reshape(1, _N, _C)


def kernel(x, edge_index, W_gat, att_src, att_dst, b_gat, W_gcn, b_gcn,
           ln_gamma, ln_beta, fc2_W, fc2_b):
    src = edge_index[0]
    dst = edge_index[1]
    xw = x @ W_gat
    a_s = xw @ att_src
    a_d = xw @ att_dst
    t_self = a_s + a_d
    ex_self = jnp.exp(jnp.where(t_self >= 0, t_self, 0.2 * t_self))
    denom = _denom_sc(src, dst, a_s, a_d).sum(0) + ex_self
    attn_self = ex_self / (denom + 1e-16)
    tab0 = jnp.concatenate([xw[:, :_QW], xw[:, _QW:2 * _QW]], axis=0)
    tab1 = jnp.concatenate([xw[:, 2 * _QW:3 * _QW], xw[:, 3 * _QW:]], axis=0)
    attn, c01 = _xq_first(src, dst, a_s, a_d, denom, tab0)
    c23 = _xq_second(src, dst, attn, tab1)
    x_enc = jnp.concatenate([c01[0], c01[1], c23[0], c23[1]], axis=1)
    x_enc = x_enc + attn_self[:, None] * xw + b_gat
    x_enc = jax.nn.elu(x_enc)
    deg = denom / (denom + 1e-16) + 1.0
    dis = jax.lax.rsqrt(deg)
    norm = dis[src] * attn * dis[dst]
    h = x_enc @ W_gcn
    comm_pre = jax.ops.segment_sum(norm[:, None] * h[src], dst, num_segments=_N)
    comm_pre = comm_pre + (attn_self * dis * dis + 1.0 / deg)[:, None] * h + b_gcn
    return _tail(comm_pre, x_enc, fc2_W, fc2_b, ln_gamma, ln_beta)


# speedup vs baseline: 18.0868x; 6.4618x over previous
import functools

import jax
import jax.numpy as jnp
from jax import lax
from jax.experimental import pallas as pl
from jax.experimental.pallas import tpu as pltpu
from jax.experimental.pallas import tpu_sc as plsc

_N = 10000
_D = 128
_C = 8
_OUT = 2
_E = 320000

_NC = 2
_NS = 16
_NW = _NC * _NS
_L = 16
_EPT = _E // _NW
_GRP = _EPT // _L

_sc_mesh = plsc.VectorSubcoreMesh(core_axis_name="c", subcore_axis_name="s")
_SC_PARAMS = pltpu.CompilerParams(
    use_tc_tiling_on_sc=False, needs_layout_passes=False)


@functools.partial(
    pl.kernel,
    out_type=jax.ShapeDtypeStruct((_NW, _N), jnp.float32),
    mesh=_sc_mesh,
    compiler_params=_SC_PARAMS,
    scratch_types=[
        pltpu.VMEM((_EPT,), jnp.int32),
        pltpu.VMEM((_EPT,), jnp.int32),
        pltpu.VMEM((_N,), jnp.float32),
        pltpu.VMEM((_N,), jnp.float32),
        pltpu.VMEM((_N,), jnp.float32),
    ],
)
def _denom_sc(src_hbm, dst_hbm, as_hbm, ad_hbm, out_hbm,
              src_v, dst_v, as_v, ad_v, acc_v):
    wid = lax.axis_index("s") * _NC + lax.axis_index("c")
    base = wid * _EPT
    pltpu.sync_copy(src_hbm.at[pl.ds(base, _EPT)], src_v)
    pltpu.sync_copy(dst_hbm.at[pl.ds(base, _EPT)], dst_v)
    pltpu.sync_copy(as_hbm, as_v)
    pltpu.sync_copy(ad_hbm, ad_v)

    zeros = jnp.zeros((_L,), jnp.float32)

    def zbody(i, c):
        acc_v[pl.ds(i * _L, _L)] = zeros
        return c

    lax.fori_loop(0, _N // _L, zbody, 0)

    def body(g, c):
        sv = src_v[pl.ds(g * _L, _L)]
        dv = dst_v[pl.ds(g * _L, _L)]
        t = plsc.load_gather(as_v, [sv]) + plsc.load_gather(ad_v, [dv])
        ex = jnp.exp(jnp.where(t >= 0, t, 0.2 * t))
        plsc.addupdate_scatter(acc_v, [dv], ex)
        return c

    lax.fori_loop(0, _GRP, body, 0)
    pltpu.sync_copy(acc_v, out_hbm.at[wid])


_QW = _D // 4
_QV = _QW // _L
_RB = 80
_EPT2 = _E // _NS
_GRP2 = _EPT2 // _L
_NB2 = _EPT2 // _RB
_RPS = _N // _NS
_NZR = _RPS // 5


def _build_xq(first):
    if first:
        out_type = (
            jax.ShapeDtypeStruct((_E,), jnp.float32),
            jax.ShapeDtypeStruct((_NC, _N, _QW), jnp.float32),
        )
        tab_scratch = [
            pltpu.VMEM((_N,), jnp.float32),
            pltpu.VMEM((_N,), jnp.float32),
            pltpu.VMEM((_N,), jnp.float32),
        ]
    else:
        out_type = jax.ShapeDtypeStruct((_NC, _N, _QW), jnp.float32)
        tab_scratch = []

    @functools.partial(
        pl.kernel,
        out_type=out_type,
        mesh=_sc_mesh,
        compiler_params=_SC_PARAMS,
        scratch_types=[
            pltpu.VMEM((_EPT2,), jnp.int32),
            pltpu.VMEM((_EPT2,), jnp.int32),
            *tab_scratch,
            pltpu.VMEM((_EPT2,), jnp.float32),
            pltpu.VMEM((_RB, _QW), jnp.float32),
            pltpu.VMEM((_NZR, _QW), jnp.float32),
            pltpu.VMEM_SHARED((_N, _QW), jnp.float32),
            pltpu.SemaphoreType.DMA,
        ],
    )
    def xq(*refs):
        if first:
            (src_hbm, dst_hbm, as_hbm, ad_hbm, den_hbm, tab_hbm,
             attn_hbm, cols_hbm,
             src_v, dst_v, as_v, ad_v, den_v, attn_v, rows_v, zbuf_v,
             acc_sh, sem) = refs
        else:
            (src_hbm, dst_hbm, attn_hbm, tab_hbm, cols_hbm,
             src_v, dst_v, attn_v, rows_v, zbuf_v, acc_sh, sem) = refs
        cid = lax.axis_index("c")
        sid = lax.axis_index("s")
        base = sid * _EPT2
        pltpu.sync_copy(src_hbm.at[pl.ds(base, _EPT2)], src_v)
        pltpu.sync_copy(dst_hbm.at[pl.ds(base, _EPT2)], dst_v)
        if first:
            pltpu.sync_copy(as_hbm, as_v)
            pltpu.sync_copy(ad_hbm, ad_v)
            pltpu.sync_copy(den_hbm, den_v)
        else:
            pltpu.sync_copy(attn_hbm.at[pl.ds(base, _EPT2)], attn_v)

        zrow = jnp.zeros((_L,), jnp.float32)
        for r in range(_NZR):
            for c in range(_QV):
                zbuf_v[r, pl.ds(c * _L, _L)] = zrow
        for k in range(5):
            pltpu.sync_copy(
                zbuf_v, acc_sh.at[pl.ds(sid * _RPS + k * _NZR, _NZR)])

        coff = cid * _N

        def prep_body(g, c):
            sv = src_v[pl.ds(g * _L, _L)]
            if first:
                dv = dst_v[pl.ds(g * _L, _L)]
                t = (plsc.load_gather(as_v, [sv])
                     + plsc.load_gather(ad_v, [dv]))
                ex = jnp.exp(jnp.where(t >= 0, t, 0.2 * t))
                attn_v[pl.ds(g * _L, _L)] = (
                    ex / (plsc.load_gather(den_v, [dv]) + 1e-16))
            src_v[pl.ds(g * _L, _L)] = sv + coff
            return c

        lax.fori_loop(0, _GRP2, prep_body, 0)

        if first:
            @pl.when(cid == 0)
            def _():
                pltpu.sync_copy(attn_v, attn_hbm.at[pl.ds(base, _EPT2)])

        plsc.subcore_barrier()

        def batch_body(b, c):
            eb = b * _RB
            pltpu.async_copy(
                tab_hbm.at[src_v.at[pl.ds(eb, _RB)]], rows_v, sem).wait()
            for g in range(_RB // _L):
                av = attn_v[pl.ds(eb + g * _L, _L)]
                for r16 in range(_L):
                    r = g * _L + r16
                    a = av[r16]
                    for cc in range(_QV):
                        rows_v[r, pl.ds(cc * _L, _L)] = (
                            rows_v[r, pl.ds(cc * _L, _L)] * a)
                dv = dst_v[pl.ds(eb + g * _L, _L)]
                pltpu.sync_copy(rows_v.at[pl.ds(g * _L, _L)], acc_sh.at[dv],
                                add=True)
            return c

        lax.fori_loop(0, _NB2, batch_body, 0)
        plsc.subcore_barrier()

        for k in range(5):
            off = sid * _RPS + k * _NZR
            pltpu.sync_copy(acc_sh.at[pl.ds(off, _NZR)], zbuf_v)
            pltpu.sync_copy(zbuf_v, cols_hbm.at[cid, pl.ds(off, _NZR)])

    return xq


_xq_first = _build_xq(True)
_xq_second = _build_xq(False)

_CW = 16


@functools.partial(
    pl.kernel,
    out_type=jax.ShapeDtypeStruct((_NC, _N, _CW), jnp.float32),
    mesh=_sc_mesh,
    compiler_params=_SC_PARAMS,
    scratch_types=[
        pltpu.VMEM((_EPT,), jnp.int32),
        pltpu.VMEM((_EPT,), jnp.int32),
        pltpu.VMEM((_EPT,), jnp.float32),
        pltpu.VMEM((_N,), jnp.float32),
        pltpu.VMEM((_RB, _CW), jnp.float32),
        pltpu.VMEM((_NZR, _CW), jnp.float32),
        pltpu.VMEM_SHARED((_N, _CW), jnp.float32),
        pltpu.SemaphoreType.DMA,
    ],
)
def _comm_sc(src_hbm, dst_hbm, attn_hbm, dis_hbm, hp_hbm, parts_hbm,
             src_v, dst_v, w_v, dis_v, rows_v, zbuf_v, acc_sh, sem):
    cid = lax.axis_index("c")
    sid = lax.axis_index("s")
    wid = sid * _NC + cid
    base = wid * _EPT
    pltpu.sync_copy(src_hbm.at[pl.ds(base, _EPT)], src_v)
    pltpu.sync_copy(dst_hbm.at[pl.ds(base, _EPT)], dst_v)
    pltpu.sync_copy(attn_hbm.at[pl.ds(base, _EPT)], w_v)
    pltpu.sync_copy(dis_hbm, dis_v)

    zrow = jnp.zeros((_L,), jnp.float32)
    for r in range(_NZR):
        zbuf_v[r, pl.ds(0, _L)] = zrow
    for k in range(5):
        pltpu.sync_copy(zbuf_v, acc_sh.at[pl.ds(sid * _RPS + k * _NZR, _NZR)])

    def norm_body(g, c):
        sv = src_v[pl.ds(g * _L, _L)]
        dv = dst_v[pl.ds(g * _L, _L)]
        w = w_v[pl.ds(g * _L, _L)]
        w_v[pl.ds(g * _L, _L)] = (
            plsc.load_gather(dis_v, [sv]) * w * plsc.load_gather(dis_v, [dv]))
        return c

    lax.fori_loop(0, _GRP, norm_body, 0)
    plsc.subcore_barrier()

    def batch_body(b, c):
        eb = b * _RB
        pltpu.async_copy(
            hp_hbm.at[src_v.at[pl.ds(eb, _RB)]], rows_v, sem).wait()
        for g in range(_RB // _L):
            wv = w_v[pl.ds(eb + g * _L, _L)]
            for r16 in range(_L):
                r = g * _L + r16
                a = wv[r16]
                rows_v[r, pl.ds(0, _L)] = rows_v[r, pl.ds(0, _L)] * a
            dv = dst_v[pl.ds(eb + g * _L, _L)]
            pltpu.sync_copy(rows_v.at[pl.ds(g * _L, _L)], acc_sh.at[dv],
                            add=True)
        return c

    lax.fori_loop(0, _EPT // _RB, batch_body, 0)
    plsc.subcore_barrier()

    for k in range(5):
        off = sid * _RPS + k * _NZR
        pltpu.sync_copy(acc_sh.at[pl.ds(off, _NZR)], zbuf_v)
        pltpu.sync_copy(zbuf_v, parts_hbm.at[cid, pl.ds(off, _NZR)])


def _tail_body(comm_ref, xenc_ref, w3_ref, g_ref, b_ref, fcb_ref,
               allcomm_ref, probs_ref):
    comm = comm_ref[...]
    comm = jnp.where(comm >= 0, comm, 0.01 * comm)
    mu = jnp.mean(comm, axis=1, keepdims=True)
    var = jnp.mean((comm - mu) ** 2, axis=1, keepdims=True)
    y = (comm - mu) * jax.lax.rsqrt(var + 1e-5) * g_ref[...] + b_ref[...]
    y = y - jnp.max(y, axis=1, keepdims=True)
    ey = jnp.exp(y)
    allcomm_ref[...] = ey / jnp.sum(ey, axis=1, keepdims=True)

    xenc = xenc_ref[...]
    logits = jnp.sum(w3_ref[...] * xenc[None, :, :], axis=(1, 2)) + fcb_ref[0, :]
    logits = logits - jnp.max(logits)
    el = jnp.exp(logits)
    probs_ref[...] = (el / jnp.sum(el))[None, :]


def _tail(comm_pre, x_enc, fc2_W, fc2_b, ln_gamma, ln_beta):
    w3 = fc2_W.reshape(_OUT, _N, _D)
    allcomm, probs = pl.pallas_call(
        _tail_body,
        out_shape=(
            jax.ShapeDtypeStruct((_N, _C), jnp.float32),
            jax.ShapeDtypeStruct((1, _OUT), jnp.float32),
        ),
    )(comm_pre, x_enc, w3, ln_gamma.reshape(1, _C), ln_beta.reshape(1, _C),
      fc2_b.reshape(1, _OUT))
    return probs, allcomm.reshape(1, _N, _C)


def kernel(x, edge_index, W_gat, att_src, att_dst, b_gat, W_gcn, b_gcn,
           ln_gamma, ln_beta, fc2_W, fc2_b):
    src = edge_index[0]
    dst = edge_index[1]
    xw = x @ W_gat
    a_s = xw @ att_src
    a_d = xw @ att_dst
    t_self = a_s + a_d
    ex_self = jnp.exp(jnp.where(t_self >= 0, t_self, 0.2 * t_self))
    denom = _denom_sc(src, dst, a_s, a_d).sum(0) + ex_self
    attn_self = ex_self / (denom + 1e-16)
    tab0 = jnp.concatenate([xw[:, :_QW], xw[:, _QW:2 * _QW]], axis=0)
    tab1 = jnp.concatenate([xw[:, 2 * _QW:3 * _QW], xw[:, 3 * _QW:]], axis=0)
    attn, c01 = _xq_first(src, dst, a_s, a_d, denom, tab0)
    c23 = _xq_second(src, dst, attn, tab1)
    x_enc = jnp.concatenate([c01[0], c01[1], c23[0], c23[1]], axis=1)
    x_enc = x_enc + attn_self[:, None] * xw + b_gat
    x_enc = jax.nn.elu(x_enc)
    deg = denom / (denom + 1e-16) + 1.0
    dis = jax.lax.rsqrt(deg)
    h = x_enc @ W_gcn
    hp = jnp.concatenate([h, jnp.zeros((_N, _CW - _C), h.dtype)], axis=1)
    parts = _comm_sc(src, dst, attn, dis, hp)
    comm_pre = parts[0, :, :_C] + parts[1, :, :_C]
    comm_pre = comm_pre + (attn_self * dis * dis + 1.0 / deg)[:, None] * h + b_gcn
    return _tail(comm_pre, x_enc, fc2_W, fc2_b, ln_gamma, ln_beta)
